# baseline (speedup 1.0000x reference)
import functools

import numpy as np

import jax
import jax.numpy as jnp
from jax import lax
from jax.experimental import pallas as pl
from jax.experimental.pallas import tpu as pltpu
from jax.experimental.pallas import tpu_sc as plsc

D_MODEL = 768
D_SAE = 32768
K_TOP = 64
B_ROWS = 4096

NW = 32
ROWS_PER_W = B_ROWS // NW
CAP = 8192
NCHUNK = D_SAE // 16
MININT = np.int32(-2147483648)


def _enc_body(x_ref, w_ref, b_ref, o_ref):
    o_ref[...] = (
        jnp.dot(x_ref[...], w_ref[...], preferred_element_type=jnp.float32)
        + b_ref[...]
    )


def _encode(x, W_enc, b_enc):
    nr = x.shape[0]
    BM, BN = 512, 2048
    return pl.pallas_call(
        _enc_body,
        grid=(nr // BM, D_SAE // BN),
        in_specs=[
            pl.BlockSpec((BM, D_MODEL), lambda i, j: (i, 0)),
            pl.BlockSpec((D_MODEL, BN), lambda i, j: (0, j)),
            pl.BlockSpec((1, BN), lambda i, j: (0, j)),
        ],
        out_specs=pl.BlockSpec((BM, BN), lambda i, j: (i, j)),
        out_shape=jax.ShapeDtypeStruct((nr, D_SAE), jnp.float32),
    )(x, W_enc, b_enc.reshape(1, D_SAE))


def _make_select_body(rpw):
    def body(pre_hbm, key_out, idx_out,
             rowbuf, ca, cb, hist, sfx, keybuf, idxbuf, sem0, sem1):
        _select_body(rpw, pre_hbm, key_out, idx_out,
                     rowbuf, ca, cb, hist, sfx, keybuf, idxbuf, sem0, sem1)
    return body


def _select_body(rpw, pre_hbm, key_out, idx_out,
                 rowbuf, ca, cb, hist, sfx, keybuf, idxbuf,
                 sem0, sem1):
    cid = lax.axis_index("c")
    sid = lax.axis_index("s")
    wid = sid * 2 + cid
    iota = lax.iota(jnp.int32, 16)
    ones = jnp.ones((16,), jnp.int32)
    zeros16 = jnp.zeros((16,), jnp.int32)

    def lanesum(ref, b):
        off = pl.multiple_of(b * 16, 16)
        return jnp.sum(ref[pl.ds(off, 16)])

    def zero_hist(nb):
        @plsc.parallel_loop(0, nb, unroll=8)
        def _(i):
            base = pl.multiple_of(i * 16, 16)
            hist[pl.ds(base, 16)] = zeros16

    def suffix_scan(nb):
        @plsc.parallel_loop(0, nb, unroll=8, carry=zeros16)
        def _(i, acc):
            off = pl.multiple_of((nb - 1 - i) * 16, 16)
            acc = acc + hist[pl.ds(off, 16)]
            sfx[pl.ds(off, 16)] = acc
            return acc

    def find_bucket(nb, t):
        def step(_, lohi):
            lo, hi = lohi
            mid = (lo + hi) // 2
            good = lanesum(sfx, mid) >= t
            return jnp.where(good, mid, lo), jnp.where(good, hi, mid)
        nsteps = max(1, nb.bit_length() - 1)
        lo, _ = lax.fori_loop(0, nsteps, step,
                              (jnp.int32(0), jnp.int32(nb)))
        s_b = lanesum(sfx, lo)
        s_next = jnp.where(lo + 1 < nb,
                           lanesum(sfx, jnp.minimum(lo + 1, nb - 1)),
                           jnp.int32(0))
        return lo, s_b - s_next, t - s_next

    def keyof(v):
        x = lax.bitcast_convert_type(v, jnp.int32)
        s = lax.shift_right_arithmetic(x, 31)
        return x ^ (s | MININT)

    def level1(rbase, t, bpred):
        zero_hist(256)
        bpv = jnp.full((16,), bpred, jnp.int32)

        @plsc.parallel_loop(0, NCHUNK, unroll=4, carry=zeros16)
        def _(c, cntv):
            off = pl.multiple_of(c * 16, 16)
            key = keyof(rowbuf[pl.ds(rbase + off, 16)])
            d = lax.shift_right_logical(key, 24)
            plsc.addupdate_scatter(hist, [d * 16 + iota], ones)
            m = d == bpv
            pos = jnp.minimum(cntv + plsc.cumsum(m.astype(jnp.int32)) - 1,
                              CAP - 1)
            plsc.store_scatter(ca, [pos], off + iota, mask=m)
            return cntv + plsc.all_reduce_population_count(m)
        suffix_scan(256)
        b1, n1, t1 = find_bucket(256, t)
        b1v = jnp.full((16,), b1, jnp.int32)
        trip = jnp.where(b1 == bpred, 0, NCHUNK)

        @plsc.parallel_loop(0, trip, unroll=4, carry=zeros16)
        def cntv(c, cntv):
            off = pl.multiple_of(c * 16, 16)
            key = keyof(rowbuf[pl.ds(rbase + off, 16)])
            m = lax.shift_right_logical(key, 24) == b1v
            pos = jnp.minimum(cntv + plsc.cumsum(m.astype(jnp.int32)) - 1,
                              CAP - 1)
            plsc.store_scatter(ca, [pos], off + iota, mask=m)
            return cntv + plsc.all_reduce_population_count(m)
        return b1, n1, t1

    def radix_level(src_c, dst_c, rbase, n, t, digit_fn, nb):
        zero_hist(nb)
        nch = (n + 15) // 16
        nv = jnp.full((16,), n, jnp.int32)
        rbv = jnp.full((16,), rbase, jnp.int32)

        def load_cand(off, valid):
            col = src_c[pl.ds(off, 16)]
            v = plsc.load_gather(rowbuf, [rbv + col], mask=valid)
            return keyof(v), 32767 - col

        @plsc.parallel_loop(0, nch, unroll=4)
        def _(c):
            off = pl.multiple_of(c * 16, 16)
            valid = (off + iota) < nv
            k, ii = load_cand(off, valid)
            d = digit_fn(k, ii)
            plsc.addupdate_scatter(hist, [d * 16 + iota], ones, mask=valid)
        suffix_scan(nb)
        bx, n_out, t_new = find_bucket(nb, t)
        bxv = jnp.full((16,), bx, jnp.int32)

        @plsc.parallel_loop(0, nch, unroll=4, carry=zeros16)
        def cntv(c, cntv):
            off = pl.multiple_of(c * 16, 16)
            valid = (off + iota) < nv
            k, ii = load_cand(off, valid)
            col = 32767 - ii
            m = valid & (digit_fn(k, ii) == bxv)
            pos = jnp.minimum(cntv + plsc.cumsum(m.astype(jnp.int32)) - 1,
                              CAP - 1)
            plsc.store_scatter(dst_c, [pos], col, mask=m)
            return cntv + plsc.all_reduce_population_count(m)
        return n_out, t_new

    def finish_small(src_c, rbase, n, t):
        nv = jnp.full((16,), n, jnp.int32)
        valid = iota < nv
        rbv = jnp.full((16,), rbase, jnp.int32)
        col = src_c[pl.ds(0, 16)]
        key = keyof(plsc.load_gather(rowbuf, [rbv + col], mask=valid))
        ii = 32767 - col
        ku = key ^ MININT
        rank = zeros16
        for j in range(16):
            jidx = jnp.full((16,), j, jnp.int32)
            colj = plsc.load_gather(src_c, [jidx])
            validj = jidx < nv
            kju = keyof(plsc.load_gather(rowbuf, [rbv + colj],
                                         mask=validj)) ^ MININT
            ij = 32767 - colj
            better = (kju > ku) | ((kju == ku) & (ij > ii))
            rank = rank + jnp.where(validj & better, 1, 0)
        sel = valid & (rank == jnp.full((16,), t - 1, jnp.int32))
        keystar = jnp.sum(jnp.where(sel, key, 0))
        idxstar = jnp.sum(jnp.where(sel, col, 0))
        return keystar, idxstar

    row0 = wid * rpw
    nrow_bytes = D_SAE
    buf0 = pl.ds(0, nrow_bytes)
    buf1 = pl.ds(nrow_bytes, nrow_bytes)

    def wait(sem, which):
        pltpu.make_async_copy(pre_hbm.at[0], rowbuf.at[which], sem).wait()

    def process(rbase, r, bpred):
        b1, n1, t1 = level1(rbase, jnp.int32(K_TOP), bpred)
        nt = (n1, t1)
        levels = [
            (lambda k, i: lax.shift_right_logical(k, 16) & 255, 256),
            (lambda k, i: lax.shift_right_logical(k, 12) & 15, 16),
            (lambda k, i: lax.shift_right_logical(k, 8) & 15, 16),
            (lambda k, i: lax.shift_right_logical(k, 4) & 15, 16),
            (lambda k, i: k & 15, 16),
            (lambda k, i: lax.shift_right_logical(i, 10) & 31, 32),
            (lambda k, i: lax.shift_right_logical(i, 5) & 31, 32),
            (lambda k, i: i & 31, 32),
        ]
        bufs = [ca, cb]
        for lv, (dfn, nb) in enumerate(levels):
            nt = radix_level(bufs[lv % 2], bufs[(lv + 1) % 2], rbase,
                             nt[0], nt[1], dfn, nb)
        keystar, idxstar = finish_small(bufs[len(levels) % 2], rbase,
                                        nt[0], nt[1])
        lane0 = iota == 0
        r16 = jnp.full((16,), r, jnp.int32)
        plsc.store_scatter(keybuf, [r16],
                           jnp.full((16,), keystar, jnp.int32), mask=lane0)
        plsc.store_scatter(idxbuf, [r16],
                           jnp.full((16,), idxstar, jnp.int32), mask=lane0)
        return b1

    pltpu.async_copy(pre_hbm.at[row0], rowbuf.at[buf0], sem0)

    def pair_body(p, bpred):
        r0 = 2 * p
        pltpu.async_copy(pre_hbm.at[row0 + r0 + 1], rowbuf.at[buf1], sem1)
        wait(sem0, buf0)
        bpred = process(0, r0, bpred)

        @pl.when(r0 + 2 < rpw)
        def _():
            pltpu.async_copy(pre_hbm.at[row0 + r0 + 2], rowbuf.at[buf0], sem0)
        wait(sem1, buf1)
        bpred = process(nrow_bytes, r0 + 1, bpred)
        return bpred

    lax.fori_loop(0, rpw // 2, pair_body, jnp.int32(-1))
    base = pl.multiple_of(wid * rpw, 8)
    pltpu.sync_copy(keybuf, key_out.at[pl.ds(base, rpw)])
    pltpu.sync_copy(idxbuf, idx_out.at[pl.ds(base, rpw)])


def _select(pre):
    nr = pre.shape[0]
    rpw = nr // NW
    mesh = plsc.VectorSubcoreMesh(core_axis_name="c", subcore_axis_name="s")
    f = pl.kernel(
        _make_select_body(rpw),
        out_type=[
            jax.ShapeDtypeStruct((nr,), jnp.int32),
            jax.ShapeDtypeStruct((nr,), jnp.int32),
        ],
        mesh=mesh,
        compiler_params=pltpu.CompilerParams(needs_layout_passes=False),
        scratch_types=[
            pltpu.VMEM((2 * D_SAE,), jnp.float32),
            pltpu.VMEM((CAP,), jnp.int32),
            pltpu.VMEM((CAP,), jnp.int32),
            pltpu.VMEM((4096,), jnp.int32),
            pltpu.VMEM((4096,), jnp.int32),
            pltpu.VMEM((rpw,), jnp.int32),
            pltpu.VMEM((rpw,), jnp.int32),
            pltpu.SemaphoreType.DMA,
            pltpu.SemaphoreType.DMA,
        ],
    )
    return f(pre)


def _dec_body(pre_ref, key_ref, cut_ref, w_ref, bd_ref, o_ref, *, bk):
    kk = pl.program_id(1)
    key = key_ref[...]
    cut = cut_ref[...]
    taub = jnp.where(key < 0, key ^ MININT, ~key)
    tau = lax.bitcast_convert_type(taub, jnp.float32)
    pre = pre_ref[...]
    col = kk * bk + lax.broadcasted_iota(jnp.int32, pre.shape, 1)
    mask = (pre > tau) | ((pre == tau) & (col <= cut))
    sp = jnp.where(mask, jnp.maximum(pre, 0.0), 0.0)
    acc = jnp.dot(sp, w_ref[...], preferred_element_type=jnp.float32)

    @pl.when(kk == 0)
    def _():
        o_ref[...] = acc + bd_ref[...]

    @pl.when(kk > 0)
    def _():
        o_ref[...] += acc


def _decode(pre, keys, cuts, W_dec, b_dec):
    nr = pre.shape[0]
    BM, BK = 512, 2048
    return pl.pallas_call(
        functools.partial(_dec_body, bk=BK),
        grid=(nr // BM, D_SAE // BK),
        in_specs=[
            pl.BlockSpec((BM, BK), lambda i, k: (i, k)),
            pl.BlockSpec((BM, 1), lambda i, k: (i, 0)),
            pl.BlockSpec((BM, 1), lambda i, k: (i, 0)),
            pl.BlockSpec((BK, D_MODEL), lambda i, k: (k, 0)),
            pl.BlockSpec((1, D_MODEL), lambda i, k: (0, 0)),
        ],
        out_specs=pl.BlockSpec((BM, D_MODEL), lambda i, k: (i, 0)),
        out_shape=jax.ShapeDtypeStruct((nr, D_MODEL), jnp.float32),
    )(pre, keys.reshape(nr, 1), cuts.reshape(nr, 1),
      W_dec, b_dec.reshape(1, D_MODEL))


NCHUNKS_PIPE = 8


def kernel(acts, W_enc, W_dec, b_enc, b_dec):
    lead = acts.shape[:-1]
    x = acts.reshape(B_ROWS, D_MODEL)
    rows = B_ROWS // NCHUNKS_PIPE
    outs = []
    for c in range(NCHUNKS_PIPE):
        xc = lax.slice_in_dim(x, c * rows, (c + 1) * rows, axis=0)
        pre = _encode(xc, W_enc, b_enc)
        keys, cuts = _select(pre)
        outs.append(_decode(pre, keys, cuts, W_dec, b_dec))
    recon = jnp.concatenate(outs, axis=0)
    return recon.reshape(lead + (D_MODEL,))


# speedup vs baseline: 1.0228x; 1.0228x over previous
import functools

import numpy as np

import jax
import jax.numpy as jnp
from jax import lax
from jax.experimental import pallas as pl
from jax.experimental.pallas import tpu as pltpu
from jax.experimental.pallas import tpu_sc as plsc

D_MODEL = 768
D_SAE = 32768
K_TOP = 64
B_ROWS = 4096

NW = 32
ROWS_PER_W = B_ROWS // NW
CAP = 8192
NCHUNK = D_SAE // 16
MININT = np.int32(-2147483648)


def _enc_body(x_ref, w_ref, b_ref, o_ref):
    o_ref[...] = (
        jnp.dot(x_ref[...], w_ref[...], preferred_element_type=jnp.float32)
        + b_ref[...]
    )


def _encode(x, W_enc, b_enc):
    nr = x.shape[0]
    BM, BN = 512, 2048
    return pl.pallas_call(
        _enc_body,
        grid=(nr // BM, D_SAE // BN),
        in_specs=[
            pl.BlockSpec((BM, D_MODEL), lambda i, j: (i, 0)),
            pl.BlockSpec((D_MODEL, BN), lambda i, j: (0, j)),
            pl.BlockSpec((1, BN), lambda i, j: (0, j)),
        ],
        out_specs=pl.BlockSpec((BM, BN), lambda i, j: (i, j)),
        out_shape=jax.ShapeDtypeStruct((nr, D_SAE), jnp.float32),
    )(x, W_enc, b_enc.reshape(1, D_SAE))


def _make_select_body(rpw):
    def body(pre_hbm, key_out, idx_out,
             rowbuf, ca, cb, hist, sfx, keybuf, idxbuf, sem0, sem1):
        _select_body(rpw, pre_hbm, key_out, idx_out,
                     rowbuf, ca, cb, hist, sfx, keybuf, idxbuf, sem0, sem1)
    return body


def _select_body(rpw, pre_hbm, key_out, idx_out,
                 rowbuf, ca, cb, hist, sfx, keybuf, idxbuf,
                 sem0, sem1):
    cid = lax.axis_index("c")
    sid = lax.axis_index("s")
    wid = sid * 2 + cid
    iota = lax.iota(jnp.int32, 16)
    ones = jnp.ones((16,), jnp.int32)
    zeros16 = jnp.zeros((16,), jnp.int32)

    def lanesum(ref, b):
        off = pl.multiple_of(b * 16, 16)
        return jnp.sum(ref[pl.ds(off, 16)])

    def zero_hist(nb):
        @plsc.parallel_loop(0, nb, unroll=8)
        def _(i):
            base = pl.multiple_of(i * 16, 16)
            hist[pl.ds(base, 16)] = zeros16

    def suffix_scan(nb):
        @plsc.parallel_loop(0, nb, unroll=8, carry=zeros16)
        def _(i, acc):
            off = pl.multiple_of((nb - 1 - i) * 16, 16)
            acc = acc + hist[pl.ds(off, 16)]
            sfx[pl.ds(off, 16)] = acc
            return acc

    def find_bucket(nb, t):
        def step(_, lohi):
            lo, hi = lohi
            mid = (lo + hi) // 2
            good = lanesum(sfx, mid) >= t
            return jnp.where(good, mid, lo), jnp.where(good, hi, mid)
        nsteps = max(1, nb.bit_length() - 1)
        lo, _ = lax.fori_loop(0, nsteps, step,
                              (jnp.int32(0), jnp.int32(nb)))
        s_b = lanesum(sfx, lo)
        s_next = jnp.where(lo + 1 < nb,
                           lanesum(sfx, jnp.minimum(lo + 1, nb - 1)),
                           jnp.int32(0))
        return lo, s_b - s_next, t - s_next

    def keyof(v):
        x = lax.bitcast_convert_type(v, jnp.int32)
        s = lax.shift_right_arithmetic(x, 31)
        return x ^ (s | MININT)

    def level1(rbase, t, bpred):
        zero_hist(256)
        bpv = jnp.full((16,), bpred, jnp.int32)

        @plsc.parallel_loop(0, NCHUNK, unroll=6, carry=zeros16)
        def _(c, cntv):
            off = pl.multiple_of(c * 16, 16)
            key = keyof(rowbuf[pl.ds(rbase + off, 16)])
            d = lax.shift_right_logical(key, 24)
            plsc.addupdate_scatter(hist, [d * 16 + iota], ones)
            m = d == bpv
            pos = jnp.minimum(cntv + plsc.cumsum(m.astype(jnp.int32)) - 1,
                              CAP - 1)
            plsc.store_scatter(ca, [pos], off + iota, mask=m)
            return cntv + plsc.all_reduce_population_count(m)
        suffix_scan(256)
        b1, n1, t1 = find_bucket(256, t)
        b1v = jnp.full((16,), b1, jnp.int32)
        trip = jnp.where(b1 == bpred, 0, NCHUNK)

        @plsc.parallel_loop(0, trip, unroll=4, carry=zeros16)
        def cntv(c, cntv):
            off = pl.multiple_of(c * 16, 16)
            key = keyof(rowbuf[pl.ds(rbase + off, 16)])
            m = lax.shift_right_logical(key, 24) == b1v
            pos = jnp.minimum(cntv + plsc.cumsum(m.astype(jnp.int32)) - 1,
                              CAP - 1)
            plsc.store_scatter(ca, [pos], off + iota, mask=m)
            return cntv + plsc.all_reduce_population_count(m)
        return b1, n1, t1

    def radix_level(src_c, dst_c, rbase, n, t, digit_fn, nb):
        zero_hist(nb)
        nch = (n + 15) // 16
        nv = jnp.full((16,), n, jnp.int32)
        rbv = jnp.full((16,), rbase, jnp.int32)

        def load_cand(off, valid):
            col = src_c[pl.ds(off, 16)]
            v = plsc.load_gather(rowbuf, [rbv + col], mask=valid)
            return keyof(v), 32767 - col

        @plsc.parallel_loop(0, nch, unroll=4)
        def _(c):
            off = pl.multiple_of(c * 16, 16)
            valid = (off + iota) < nv
            k, ii = load_cand(off, valid)
            d = digit_fn(k, ii)
            plsc.addupdate_scatter(hist, [d * 16 + iota], ones, mask=valid)
        suffix_scan(nb)
        bx, n_out, t_new = find_bucket(nb, t)
        bxv = jnp.full((16,), bx, jnp.int32)

        @plsc.parallel_loop(0, nch, unroll=4, carry=zeros16)
        def cntv(c, cntv):
            off = pl.multiple_of(c * 16, 16)
            valid = (off + iota) < nv
            k, ii = load_cand(off, valid)
            col = 32767 - ii
            m = valid & (digit_fn(k, ii) == bxv)
            pos = jnp.minimum(cntv + plsc.cumsum(m.astype(jnp.int32)) - 1,
                              CAP - 1)
            plsc.store_scatter(dst_c, [pos], col, mask=m)
            return cntv + plsc.all_reduce_population_count(m)
        return n_out, t_new

    def finish_small(src_c, rbase, n, t):
        nv = jnp.full((16,), n, jnp.int32)
        valid = iota < nv
        rbv = jnp.full((16,), rbase, jnp.int32)
        col = src_c[pl.ds(0, 16)]
        key = keyof(plsc.load_gather(rowbuf, [rbv + col], mask=valid))
        ii = 32767 - col
        ku = key ^ MININT
        rank = zeros16
        for j in range(16):
            jidx = jnp.full((16,), j, jnp.int32)
            colj = plsc.load_gather(src_c, [jidx])
            validj = jidx < nv
            kju = keyof(plsc.load_gather(rowbuf, [rbv + colj],
                                         mask=validj)) ^ MININT
            ij = 32767 - colj
            better = (kju > ku) | ((kju == ku) & (ij > ii))
            rank = rank + jnp.where(validj & better, 1, 0)
        sel = valid & (rank == jnp.full((16,), t - 1, jnp.int32))
        keystar = jnp.sum(jnp.where(sel, key, 0))
        idxstar = jnp.sum(jnp.where(sel, col, 0))
        return keystar, idxstar

    row0 = wid * rpw
    nrow_bytes = D_SAE
    buf0 = pl.ds(0, nrow_bytes)
    buf1 = pl.ds(nrow_bytes, nrow_bytes)

    def wait(sem, which):
        pltpu.make_async_copy(pre_hbm.at[0], rowbuf.at[which], sem).wait()

    def process(rbase, r, bpred):
        b1, n1, t1 = level1(rbase, jnp.int32(K_TOP), bpred)
        nt = (n1, t1)
        levels = [
            (lambda k, i: lax.shift_right_logical(k, 16) & 255, 256),
            (lambda k, i: lax.shift_right_logical(k, 12) & 15, 16),
            (lambda k, i: lax.shift_right_logical(k, 8) & 15, 16),
            (lambda k, i: lax.shift_right_logical(k, 4) & 15, 16),
            (lambda k, i: k & 15, 16),
            (lambda k, i: lax.shift_right_logical(i, 10) & 31, 32),
            (lambda k, i: lax.shift_right_logical(i, 5) & 31, 32),
            (lambda k, i: i & 31, 32),
        ]
        bufs = [ca, cb]
        for lv, (dfn, nb) in enumerate(levels):
            nt = radix_level(bufs[lv % 2], bufs[(lv + 1) % 2], rbase,
                             nt[0], nt[1], dfn, nb)
        keystar, idxstar = finish_small(bufs[len(levels) % 2], rbase,
                                        nt[0], nt[1])
        lane0 = iota == 0
        r16 = jnp.full((16,), r, jnp.int32)
        plsc.store_scatter(keybuf, [r16],
                           jnp.full((16,), keystar, jnp.int32), mask=lane0)
        plsc.store_scatter(idxbuf, [r16],
                           jnp.full((16,), idxstar, jnp.int32), mask=lane0)
        return b1

    pltpu.async_copy(pre_hbm.at[row0], rowbuf.at[buf0], sem0)

    def pair_body(p, bpred):
        r0 = 2 * p
        pltpu.async_copy(pre_hbm.at[row0 + r0 + 1], rowbuf.at[buf1], sem1)
        wait(sem0, buf0)
        bpred = process(0, r0, bpred)

        @pl.when(r0 + 2 < rpw)
        def _():
            pltpu.async_copy(pre_hbm.at[row0 + r0 + 2], rowbuf.at[buf0], sem0)
        wait(sem1, buf1)
        bpred = process(nrow_bytes, r0 + 1, bpred)
        return bpred

    lax.fori_loop(0, rpw // 2, pair_body, jnp.int32(-1))
    base = pl.multiple_of(wid * rpw, 8)
    pltpu.sync_copy(keybuf, key_out.at[pl.ds(base, rpw)])
    pltpu.sync_copy(idxbuf, idx_out.at[pl.ds(base, rpw)])


def _select(pre):
    nr = pre.shape[0]
    rpw = nr // NW
    mesh = plsc.VectorSubcoreMesh(core_axis_name="c", subcore_axis_name="s")
    f = pl.kernel(
        _make_select_body(rpw),
        out_type=[
            jax.ShapeDtypeStruct((nr,), jnp.int32),
            jax.ShapeDtypeStruct((nr,), jnp.int32),
        ],
        mesh=mesh,
        compiler_params=pltpu.CompilerParams(needs_layout_passes=False),
        scratch_types=[
            pltpu.VMEM((2 * D_SAE,), jnp.float32),
            pltpu.VMEM((CAP,), jnp.int32),
            pltpu.VMEM((CAP,), jnp.int32),
            pltpu.VMEM((4096,), jnp.int32),
            pltpu.VMEM((4096,), jnp.int32),
            pltpu.VMEM((rpw,), jnp.int32),
            pltpu.VMEM((rpw,), jnp.int32),
            pltpu.SemaphoreType.DMA,
            pltpu.SemaphoreType.DMA,
        ],
    )
    return f(pre)


def _dec_body(pre_ref, key_ref, cut_ref, w_ref, bd_ref, o_ref, *, bk):
    kk = pl.program_id(1)
    key = key_ref[...]
    cut = cut_ref[...]
    taub = jnp.where(key < 0, key ^ MININT, ~key)
    tau = lax.bitcast_convert_type(taub, jnp.float32)
    pre = pre_ref[...]
    col = kk * bk + lax.broadcasted_iota(jnp.int32, pre.shape, 1)
    mask = (pre > tau) | ((pre == tau) & (col <= cut))
    sp = jnp.where(mask, jnp.maximum(pre, 0.0), 0.0)
    acc = jnp.dot(sp, w_ref[...], preferred_element_type=jnp.float32)

    @pl.when(kk == 0)
    def _():
        o_ref[...] = acc + bd_ref[...]

    @pl.when(kk > 0)
    def _():
        o_ref[...] += acc


def _decode(pre, keys, cuts, W_dec, b_dec):
    nr = pre.shape[0]
    BM, BK = 512, 2048
    return pl.pallas_call(
        functools.partial(_dec_body, bk=BK),
        grid=(nr // BM, D_SAE // BK),
        in_specs=[
            pl.BlockSpec((BM, BK), lambda i, k: (i, k)),
            pl.BlockSpec((BM, 1), lambda i, k: (i, 0)),
            pl.BlockSpec((BM, 1), lambda i, k: (i, 0)),
            pl.BlockSpec((BK, D_MODEL), lambda i, k: (k, 0)),
            pl.BlockSpec((1, D_MODEL), lambda i, k: (0, 0)),
        ],
        out_specs=pl.BlockSpec((BM, D_MODEL), lambda i, k: (i, 0)),
        out_shape=jax.ShapeDtypeStruct((nr, D_MODEL), jnp.float32),
    )(pre, keys.reshape(nr, 1), cuts.reshape(nr, 1),
      W_dec, b_dec.reshape(1, D_MODEL))


NCHUNKS_PIPE = 4


def kernel(acts, W_enc, W_dec, b_enc, b_dec):
    lead = acts.shape[:-1]
    x = acts.reshape(B_ROWS, D_MODEL)
    rows = B_ROWS // NCHUNKS_PIPE
    outs = []
    for c in range(NCHUNKS_PIPE):
        xc = lax.slice_in_dim(x, c * rows, (c + 1) * rows, axis=0)
        pre = _encode(xc, W_enc, b_enc)
        keys, cuts = _select(pre)
        outs.append(_decode(pre, keys, cuts, W_dec, b_dec))
    recon = jnp.concatenate(outs, axis=0)
    return recon.reshape(lead + (D_MODEL,))


# speedup vs baseline: 1.0429x; 1.0197x over previous
import functools

import numpy as np

import jax
import jax.numpy as jnp
from jax import lax
from jax.experimental import pallas as pl
from jax.experimental.pallas import tpu as pltpu
from jax.experimental.pallas import tpu_sc as plsc

D_MODEL = 768
D_SAE = 32768
K_TOP = 64
B_ROWS = 4096

NW = 32
ROWS_PER_W = B_ROWS // NW
CAP = 8192
NCHUNK = D_SAE // 16
MININT = np.int32(-2147483648)


def _enc_body(x_ref, w_ref, b_ref, o_ref):
    o_ref[...] = (
        jnp.dot(x_ref[...], w_ref[...], preferred_element_type=jnp.float32)
        + b_ref[...]
    )


def _encode(x, W_enc, b_enc):
    nr = x.shape[0]
    BM, BN = 512, 2048
    return pl.pallas_call(
        _enc_body,
        grid=(nr // BM, D_SAE // BN),
        in_specs=[
            pl.BlockSpec((BM, D_MODEL), lambda i, j: (i, 0)),
            pl.BlockSpec((D_MODEL, BN), lambda i, j: (0, j)),
            pl.BlockSpec((1, BN), lambda i, j: (0, j)),
        ],
        out_specs=pl.BlockSpec((BM, BN), lambda i, j: (i, j)),
        out_shape=jax.ShapeDtypeStruct((nr, D_SAE), jnp.float32),
    )(x, W_enc, b_enc.reshape(1, D_SAE))


def _make_select_body(rpw):
    def body(pre_hbm, key_out, idx_out,
             rowbuf, ca, cb, hist, sfx, keybuf, idxbuf, sem0, sem1):
        _select_body(rpw, pre_hbm, key_out, idx_out,
                     rowbuf, ca, cb, hist, sfx, keybuf, idxbuf, sem0, sem1)
    return body


def _select_body(rpw, pre_hbm, key_out, idx_out,
                 rowbuf, ca, cb, hist, sfx, keybuf, idxbuf,
                 sem0, sem1):
    cid = lax.axis_index("c")
    sid = lax.axis_index("s")
    wid = sid * 2 + cid
    iota = lax.iota(jnp.int32, 16)
    ones = jnp.ones((16,), jnp.int32)
    zeros16 = jnp.zeros((16,), jnp.int32)

    def lanesum(ref, b):
        off = pl.multiple_of(b * 16, 16)
        return jnp.sum(ref[pl.ds(off, 16)])

    def zero_hist(nb):
        @plsc.parallel_loop(0, nb, unroll=8)
        def _(i):
            base = pl.multiple_of(i * 16, 16)
            hist[pl.ds(base, 16)] = zeros16

    def suffix_scan(nb):
        @plsc.parallel_loop(0, nb, unroll=8, carry=zeros16)
        def _(i, acc):
            off = pl.multiple_of((nb - 1 - i) * 16, 16)
            acc = acc + hist[pl.ds(off, 16)]
            sfx[pl.ds(off, 16)] = acc
            return acc

    def find_bucket(nb, t):
        def step(_, lohi):
            lo, hi = lohi
            mid = (lo + hi) // 2
            good = lanesum(sfx, mid) >= t
            return jnp.where(good, mid, lo), jnp.where(good, hi, mid)
        nsteps = max(1, nb.bit_length() - 1)
        lo, _ = lax.fori_loop(0, nsteps, step,
                              (jnp.int32(0), jnp.int32(nb)))
        s_b = lanesum(sfx, lo)
        s_next = jnp.where(lo + 1 < nb,
                           lanesum(sfx, jnp.minimum(lo + 1, nb - 1)),
                           jnp.int32(0))
        return lo, s_b - s_next, t - s_next

    def keyof(v):
        x = lax.bitcast_convert_type(v, jnp.int32)
        s = lax.shift_right_arithmetic(x, 31)
        return x ^ (s | MININT)

    def level1(rbase, t, bpred):
        zero_hist(256)
        bpv = jnp.full((16,), bpred, jnp.int32)

        @plsc.parallel_loop(0, NCHUNK, unroll=6, carry=zeros16)
        def _(c, cntv):
            off = pl.multiple_of(c * 16, 16)
            key = keyof(rowbuf[pl.ds(rbase + off, 16)])
            d = lax.shift_right_logical(key, 24)
            plsc.addupdate_scatter(hist, [lax.shift_left(d, 4) | iota], ones)
            m = d == bpv
            pos = jnp.minimum(cntv + plsc.cumsum(ones, mask=m) - 1,
                              CAP - 1)
            plsc.store_scatter(ca, [pos], off + iota, mask=m)
            return cntv + plsc.all_reduce_population_count(m)
        suffix_scan(256)
        b1, n1, t1 = find_bucket(256, t)
        b1v = jnp.full((16,), b1, jnp.int32)
        trip = jnp.where(b1 == bpred, 0, NCHUNK)

        @plsc.parallel_loop(0, trip, unroll=4, carry=zeros16)
        def cntv(c, cntv):
            off = pl.multiple_of(c * 16, 16)
            key = keyof(rowbuf[pl.ds(rbase + off, 16)])
            m = lax.shift_right_logical(key, 24) == b1v
            pos = jnp.minimum(cntv + plsc.cumsum(ones, mask=m) - 1,
                              CAP - 1)
            plsc.store_scatter(ca, [pos], off + iota, mask=m)
            return cntv + plsc.all_reduce_population_count(m)
        return b1, n1, t1

    def radix_level(src_c, dst_c, rbase, n, t, digit_fn, nb):
        zero_hist(nb)
        nch = (n + 15) // 16
        nv = jnp.full((16,), n, jnp.int32)
        rbv = jnp.full((16,), rbase, jnp.int32)

        def load_cand(off, valid):
            col = src_c[pl.ds(off, 16)]
            v = plsc.load_gather(rowbuf, [rbv + col], mask=valid)
            return keyof(v), 32767 - col

        @plsc.parallel_loop(0, nch, unroll=4)
        def _(c):
            off = pl.multiple_of(c * 16, 16)
            valid = (off + iota) < nv
            k, ii = load_cand(off, valid)
            d = digit_fn(k, ii)
            plsc.addupdate_scatter(hist, [lax.shift_left(d, 4) | iota], ones, mask=valid)
        suffix_scan(nb)
        bx, n_out, t_new = find_bucket(nb, t)
        bxv = jnp.full((16,), bx, jnp.int32)

        @plsc.parallel_loop(0, nch, unroll=4, carry=zeros16)
        def cntv(c, cntv):
            off = pl.multiple_of(c * 16, 16)
            valid = (off + iota) < nv
            k, ii = load_cand(off, valid)
            col = 32767 - ii
            m = valid & (digit_fn(k, ii) == bxv)
            pos = jnp.minimum(cntv + plsc.cumsum(ones, mask=m) - 1,
                              CAP - 1)
            plsc.store_scatter(dst_c, [pos], col, mask=m)
            return cntv + plsc.all_reduce_population_count(m)
        return n_out, t_new

    def finish_small(src_c, rbase, n, t):
        nv = jnp.full((16,), n, jnp.int32)
        valid = iota < nv
        rbv = jnp.full((16,), rbase, jnp.int32)
        col = src_c[pl.ds(0, 16)]
        key = keyof(plsc.load_gather(rowbuf, [rbv + col], mask=valid))
        ii = 32767 - col
        ku = key ^ MININT
        rank = zeros16
        for j in range(16):
            jidx = jnp.full((16,), j, jnp.int32)
            colj = plsc.load_gather(src_c, [jidx])
            validj = jidx < nv
            kju = keyof(plsc.load_gather(rowbuf, [rbv + colj],
                                         mask=validj)) ^ MININT
            ij = 32767 - colj
            better = (kju > ku) | ((kju == ku) & (ij > ii))
            rank = rank + jnp.where(validj & better, 1, 0)
        sel = valid & (rank == jnp.full((16,), t - 1, jnp.int32))
        keystar = jnp.sum(jnp.where(sel, key, 0))
        idxstar = jnp.sum(jnp.where(sel, col, 0))
        return keystar, idxstar

    row0 = wid * rpw
    nrow_bytes = D_SAE
    buf0 = pl.ds(0, nrow_bytes)
    buf1 = pl.ds(nrow_bytes, nrow_bytes)

    def wait(sem, which):
        pltpu.make_async_copy(pre_hbm.at[0], rowbuf.at[which], sem).wait()

    def process(rbase, r, bpred):
        b1, n1, t1 = level1(rbase, jnp.int32(K_TOP), bpred)
        nt = (n1, t1)
        levels = [
            (lambda k, i: lax.shift_right_logical(k, 16) & 255, 256),
            (lambda k, i: lax.shift_right_logical(k, 12) & 15, 16),
            (lambda k, i: lax.shift_right_logical(k, 8) & 15, 16),
            (lambda k, i: lax.shift_right_logical(k, 4) & 15, 16),
            (lambda k, i: k & 15, 16),
            (lambda k, i: lax.shift_right_logical(i, 10) & 31, 32),
            (lambda k, i: lax.shift_right_logical(i, 5) & 31, 32),
            (lambda k, i: i & 31, 32),
        ]
        bufs = [ca, cb]
        for lv, (dfn, nb) in enumerate(levels):
            nt = radix_level(bufs[lv % 2], bufs[(lv + 1) % 2], rbase,
                             nt[0], nt[1], dfn, nb)
        keystar, idxstar = finish_small(bufs[len(levels) % 2], rbase,
                                        nt[0], nt[1])
        lane0 = iota == 0
        r16 = jnp.full((16,), r, jnp.int32)
        plsc.store_scatter(keybuf, [r16],
                           jnp.full((16,), keystar, jnp.int32), mask=lane0)
        plsc.store_scatter(idxbuf, [r16],
                           jnp.full((16,), idxstar, jnp.int32), mask=lane0)
        return b1

    pltpu.async_copy(pre_hbm.at[row0], rowbuf.at[buf0], sem0)

    def pair_body(p, bpred):
        r0 = 2 * p
        pltpu.async_copy(pre_hbm.at[row0 + r0 + 1], rowbuf.at[buf1], sem1)
        wait(sem0, buf0)
        bpred = process(0, r0, bpred)

        @pl.when(r0 + 2 < rpw)
        def _():
            pltpu.async_copy(pre_hbm.at[row0 + r0 + 2], rowbuf.at[buf0], sem0)
        wait(sem1, buf1)
        bpred = process(nrow_bytes, r0 + 1, bpred)
        return bpred

    lax.fori_loop(0, rpw // 2, pair_body, jnp.int32(-1))
    base = pl.multiple_of(wid * rpw, 8)
    pltpu.sync_copy(keybuf, key_out.at[pl.ds(base, rpw)])
    pltpu.sync_copy(idxbuf, idx_out.at[pl.ds(base, rpw)])


def _select(pre):
    nr = pre.shape[0]
    rpw = nr // NW
    mesh = plsc.VectorSubcoreMesh(core_axis_name="c", subcore_axis_name="s")
    f = pl.kernel(
        _make_select_body(rpw),
        out_type=[
            jax.ShapeDtypeStruct((nr,), jnp.int32),
            jax.ShapeDtypeStruct((nr,), jnp.int32),
        ],
        mesh=mesh,
        compiler_params=pltpu.CompilerParams(needs_layout_passes=False),
        scratch_types=[
            pltpu.VMEM((2 * D_SAE,), jnp.float32),
            pltpu.VMEM((CAP,), jnp.int32),
            pltpu.VMEM((CAP,), jnp.int32),
            pltpu.VMEM((4096,), jnp.int32),
            pltpu.VMEM((4096,), jnp.int32),
            pltpu.VMEM((rpw,), jnp.int32),
            pltpu.VMEM((rpw,), jnp.int32),
            pltpu.SemaphoreType.DMA,
            pltpu.SemaphoreType.DMA,
        ],
    )
    return f(pre)


def _dec_body(pre_ref, key_ref, cut_ref, w_ref, bd_ref, o_ref, *, bk):
    kk = pl.program_id(1)
    key = key_ref[...]
    cut = cut_ref[...]
    taub = jnp.where(key < 0, key ^ MININT, ~key)
    tau = lax.bitcast_convert_type(taub, jnp.float32)
    pre = pre_ref[...]
    col = kk * bk + lax.broadcasted_iota(jnp.int32, pre.shape, 1)
    mask = (pre > tau) | ((pre == tau) & (col <= cut))
    sp = jnp.where(mask, jnp.maximum(pre, 0.0), 0.0)
    acc = jnp.dot(sp, w_ref[...], preferred_element_type=jnp.float32)

    @pl.when(kk == 0)
    def _():
        o_ref[...] = acc + bd_ref[...]

    @pl.when(kk > 0)
    def _():
        o_ref[...] += acc


def _decode(pre, keys, cuts, W_dec, b_dec):
    nr = pre.shape[0]
    BM, BK = 512, 2048
    return pl.pallas_call(
        functools.partial(_dec_body, bk=BK),
        grid=(nr // BM, D_SAE // BK),
        in_specs=[
            pl.BlockSpec((BM, BK), lambda i, k: (i, k)),
            pl.BlockSpec((BM, 1), lambda i, k: (i, 0)),
            pl.BlockSpec((BM, 1), lambda i, k: (i, 0)),
            pl.BlockSpec((BK, D_MODEL), lambda i, k: (k, 0)),
            pl.BlockSpec((1, D_MODEL), lambda i, k: (0, 0)),
        ],
        out_specs=pl.BlockSpec((BM, D_MODEL), lambda i, k: (i, 0)),
        out_shape=jax.ShapeDtypeStruct((nr, D_MODEL), jnp.float32),
    )(pre, keys.reshape(nr, 1), cuts.reshape(nr, 1),
      W_dec, b_dec.reshape(1, D_MODEL))


NCHUNKS_PIPE = 4


def kernel(acts, W_enc, W_dec, b_enc, b_dec):
    lead = acts.shape[:-1]
    x = acts.reshape(B_ROWS, D_MODEL)
    rows = B_ROWS // NCHUNKS_PIPE
    outs = []
    for c in range(NCHUNKS_PIPE):
        xc = lax.slice_in_dim(x, c * rows, (c + 1) * rows, axis=0)
        pre = _encode(xc, W_enc, b_enc)
        keys, cuts = _select(pre)
        outs.append(_decode(pre, keys, cuts, W_dec, b_dec))
    recon = jnp.concatenate(outs, axis=0)
    return recon.reshape(lead + (D_MODEL,))
